# Initial kernel scaffold; baseline (speedup 1.0000x reference)
import jax
import jax.numpy as jnp
from jax.experimental import pallas as pl


def kernel(x, edge_index, edge_attr, batch, W1, b1, bn_gamma, bn_beta, bn_mean, bn_var, W2, b2, ln_w, ln_b, gn_w, gn_b, gn_scale):
    raise NotImplementedError("write your pallas kernel here")


# speedup vs baseline: 3.1176x; 3.1176x over previous
import functools

import jax
import jax.numpy as jnp
from jax import lax
from jax.experimental import pallas as pl
from jax.experimental.pallas import tpu as pltpu
from jax.experimental.pallas import tpu_sc as plsc

N = 10000
E = 160000
D = 256
B = 64
EPS = 1e-5

NC = 2
NS = 16
LANES = 16
HALF = D // NC

EPT = E // NS
CHUNK = 80
NCHUNK = EPT // CHUNK
ROWS = N // NS
WCHUNK = 125
NWC = ROWS // WCHUNK

NB = 400
NBLK = N // NB


def _sc_body(src_hbm, dst_hbm, x2_hbm, ea2_hbm, out_hbm,
             src_v, dst_v, xi_v, ei_v, xrows_v, ea_v, zrow_v, acc_sh,
             sem_x, sem_e):
    c = lax.axis_index("c")
    s = lax.axis_index("s")

    zero16 = jnp.zeros((LANES,), jnp.float32)

    def zrow(r, carry):
        for j in range(HALF // LANES):
            zrow_v[r, pl.ds(j * LANES, LANES)] = zero16
        return carry

    lax.fori_loop(0, WCHUNK, zrow, 0)
    row0 = s * ROWS
    for k in range(NWC):
        pltpu.sync_copy(zrow_v, acc_sh.at[pl.ds(row0 + k * WCHUNK, WCHUNK)])
    plsc.subcore_barrier()

    lane2 = lax.iota(jnp.int32, LANES) * 2
    e0 = s * EPT

    def chunk_body(k, carry):
        base = e0 + k * CHUNK
        pltpu.sync_copy(src_hbm.at[pl.ds(base, CHUNK)], src_v)
        pltpu.sync_copy(dst_hbm.at[pl.ds(base, CHUNK)], dst_v)
        for q in range(CHUNK // LANES):
            sl = pl.ds(q * LANES, LANES)
            xi_v[sl] = src_v[sl] * 2 + c
            ei_v[sl] = lane2 + (2 * (base + q * LANES) + c)
        cp_x = pltpu.async_copy(x2_hbm.at[xi_v], xrows_v, sem_x)
        cp_e = pltpu.async_copy(ea2_hbm.at[ei_v], ea_v, sem_e)
        cp_x.wait()
        cp_e.wait()

        def rowf(r, rc):
            for j in range(HALF // LANES):
                sl = pl.ds(j * LANES, LANES)
                xrows_v[r, sl] = jnp.maximum(xrows_v[r, sl] + ea_v[r, sl], 0.0)
            return rc

        lax.fori_loop(0, CHUNK, rowf, 0)
        pltpu.sync_copy(xrows_v, acc_sh.at[dst_v], add=True)
        return carry

    lax.fori_loop(0, NCHUNK, chunk_body, 0)
    plsc.subcore_barrier()

    for k in range(NWC):
        sl = pl.ds(row0 + k * WCHUNK, WCHUNK)
        pltpu.sync_copy(acc_sh.at[sl], out_hbm.at[c, sl])


@functools.lru_cache(maxsize=None)
def _build_sc_aggregate():
    return pl.kernel(
        _sc_body,
        out_type=jax.ShapeDtypeStruct((NC, N, HALF), jnp.float32),
        mesh=plsc.VectorSubcoreMesh(
            core_axis_name="c", subcore_axis_name="s",
            num_cores=NC, num_subcores=NS),
        scratch_types=[
            pltpu.VMEM((CHUNK,), jnp.int32),
            pltpu.VMEM((CHUNK,), jnp.int32),
            pltpu.VMEM((CHUNK,), jnp.int32),
            pltpu.VMEM((CHUNK,), jnp.int32),
            pltpu.VMEM((CHUNK, HALF), jnp.float32),
            pltpu.VMEM((CHUNK, HALF), jnp.float32),
            pltpu.VMEM((WCHUNK, HALF), jnp.float32),
            pltpu.VMEM_SHARED((N, HALF), jnp.float32),
            pltpu.SemaphoreType.DMA,
            pltpu.SemaphoreType.DMA,
        ],
        compiler_params=pltpu.CompilerParams(use_tc_tiling_on_sc=False),
    )


def _sc_aggregate(src, dst, x2, ea2):
    return _build_sc_aggregate()(src, dst, x2, ea2)


def _mlp_stats_kernel(x_ref, agg_ref, batch_ref, w1_ref, b1_ref, g_ref,
                      be_ref, mu_ref, va_ref, w2_ref, b2_ref,
                      out_ref, stats_ref):
    i = pl.program_id(0)
    x = x_ref[...]
    h = x + jnp.concatenate([agg_ref[0], agg_ref[1]], axis=1)
    h1 = jnp.dot(h, w1_ref[...], preferred_element_type=jnp.float32)
    scale = g_ref[...] * lax.rsqrt(va_ref[...] + EPS)
    h1 = (h1 + b1_ref[...] - mu_ref[...]) * scale + be_ref[...]
    h1 = jnp.maximum(h1, 0.0)
    out = jnp.dot(h1, w2_ref[...], preferred_element_type=jnp.float32)
    out = out + b2_ref[...]
    out_ref[...] = out

    batch_col = batch_ref[0, 0, :].reshape(NB, 1)
    iota_b = lax.broadcasted_iota(jnp.int32, (NB, B), 1)
    p = (batch_col == iota_b).astype(jnp.float32)
    m1 = lax.dot_general(p, out, (((0,), (0,)), ((), ())),
                         preferred_element_type=jnp.float32)
    m2 = lax.dot_general(p, out * out, (((0,), (0,)), ((), ())),
                         preferred_element_type=jnp.float32)
    deg = jnp.broadcast_to(jnp.sum(p, axis=0)[:, None], (B, D))
    stacked = jnp.stack([m1, m2, deg])

    @pl.when(i == 0)
    def _():
        stats_ref[...] = stacked

    @pl.when(i > 0)
    def _():
        stats_ref[...] = stats_ref[...] + stacked


def _final_kernel(x_ref, out_in_ref, batch_ref, stats_ref, lnw_ref, lnb_ref,
                  gnw_ref, gnb_ref, gns_ref, res_ref):
    m1 = stats_ref[0]
    m2 = stats_ref[1]
    deg = stats_ref[2, :, 0:1]
    cnt = jnp.maximum(deg, 1.0)
    norm = cnt * D
    ms1 = jnp.sum(m1, axis=1, keepdims=True)
    ms2 = jnp.sum(m2, axis=1, keepdims=True)
    m = ms1 / norm
    varb = ms2 / norm - m * m
    inv_s = lax.rsqrt(varb + EPS)
    lnw = lnw_ref[...][None, :]
    gns = gns_ref[...][None, :]
    gnw = gnw_ref[...][None, :]
    a = lnw * inv_s
    cc = lnb_ref[...][None, :] - m * a
    mu1 = m1 / cnt
    mu2 = m2 / cnt
    beta = cc * (1.0 - gns) - a * mu1 * gns
    gvar = a * a * mu2 + 2.0 * a * beta * mu1 + beta * beta
    invt = lax.rsqrt(gvar + EPS)
    gamma = gnw * a * invt
    delta = gnw * beta * invt + gnb_ref[...][None, :]

    batch_col = batch_ref[0, 0, :].reshape(NB, 1)
    iota_b = lax.broadcasted_iota(jnp.int32, (NB, B), 1)
    p = (batch_col == iota_b).astype(jnp.float32)
    gn = jnp.dot(p, gamma, preferred_element_type=jnp.float32)
    dn = jnp.dot(p, delta, preferred_element_type=jnp.float32)
    res_ref[...] = x_ref[...] + jnp.maximum(gn * out_in_ref[...] + dn, 0.0)


def _full(shape):
    nd = len(shape)
    return pl.BlockSpec(shape, lambda i: (0,) * nd)


def kernel(x, edge_index, edge_attr, batch, W1, b1, bn_gamma, bn_beta,
           bn_mean, bn_var, W2, b2, ln_w, ln_b, gn_w, gn_b, gn_scale):
    src = edge_index[0]
    dst = edge_index[1]
    x2 = x.reshape(2 * N, HALF)
    ea2 = edge_attr.reshape(2 * E, HALF)
    agg = _sc_aggregate(src, dst, x2, ea2)

    batch3 = batch.reshape(NBLK, 1, NB)
    blk = pl.BlockSpec((NB, D), lambda i: (i, 0))
    bblk = pl.BlockSpec((1, 1, NB), lambda i: (i, 0, 0))

    out, stats = pl.pallas_call(
        _mlp_stats_kernel,
        grid=(NBLK,),
        in_specs=[
            blk,
            pl.BlockSpec((NC, NB, HALF), lambda i: (0, i, 0)),
            bblk,
            _full((D, 2 * D)), _full((2 * D,)), _full((2 * D,)),
            _full((2 * D,)), _full((2 * D,)), _full((2 * D,)),
            _full((2 * D, D)), _full((D,)),
        ],
        out_specs=[
            blk,
            pl.BlockSpec((3, B, D), lambda i: (0, 0, 0)),
        ],
        out_shape=[
            jax.ShapeDtypeStruct((N, D), jnp.float32),
            jax.ShapeDtypeStruct((3, B, D), jnp.float32),
        ],
        compiler_params=pltpu.CompilerParams(
            dimension_semantics=("arbitrary",)),
    )(x, agg, batch3, W1, b1, bn_gamma, bn_beta, bn_mean, bn_var, W2, b2)

    res = pl.pallas_call(
        _final_kernel,
        grid=(NBLK,),
        in_specs=[
            blk, blk, bblk, _full((3, B, D)),
            _full((D,)), _full((D,)), _full((D,)), _full((D,)), _full((D,)),
        ],
        out_specs=blk,
        out_shape=jax.ShapeDtypeStruct((N, D), jnp.float32),
        compiler_params=pltpu.CompilerParams(
            dimension_semantics=("arbitrary",)),
    )(x, out, batch3, stats, ln_w, ln_b, gn_w, gn_b, gn_scale)
    return res
